# Initial kernel scaffold; baseline (speedup 1.0000x reference)
import functools

import jax
import jax.numpy as jnp
from jax import lax
from jax.experimental import pallas as pl
from jax.experimental.pallas import tpu as pltpu
from jax.experimental.pallas import tpu_sc as plsc

_T, _D, _F, _E = 2048, 1024, 2048, 64
_RB = 32
_FB = 512
_NFB = _F // _FB
_TB = 256
_NW = 32
_BPW = _T // _NW


def _router_body(x_ref, rw_ref, xs_ref, pos_ref, offs_ref, cnts_ref, loss_ref):
    x = x_ref[...]
    logits = jnp.dot(x, rw_ref[...], preferred_element_type=jnp.float32)
    m = jnp.max(logits, axis=1, keepdims=True)
    eidx = lax.broadcasted_iota(jnp.int32, (_T, _E), 1)
    idx = jnp.min(jnp.where(logits == m, eidx, _E), axis=1, keepdims=True)
    onehot = (eidx == idx).astype(jnp.float32)
    score = jax.nn.sigmoid(m)

    counts = jnp.sum(onehot, axis=0, keepdims=True)
    scoresum = jnp.sum(onehot * score, axis=0, keepdims=True)
    ri = lax.broadcasted_iota(jnp.int32, (_E, _E), 0)
    ci = lax.broadcasted_iota(jnp.int32, (_E, _E), 1)
    offs = jnp.dot(counts, (ri < ci).astype(jnp.float32),
                   preferred_element_type=jnp.float32)

    rt = lax.broadcasted_iota(jnp.int32, (_TB, _TB), 0)
    ct = lax.broadcasted_iota(jnp.int32, (_TB, _TB), 1)
    lstrict = (ct < rt).astype(jnp.float32)
    carry = jnp.zeros((1, _E), jnp.float32)
    for b in range(_T // _TB):
        mb = lax.slice(onehot, (b * _TB, 0), ((b + 1) * _TB, _E))
        rank_b = jnp.dot(lstrict, mb, preferred_element_type=jnp.float32) + carry
        pos_b = jnp.sum(mb * (rank_b + offs), axis=1)
        pos_ref[b * 2:(b + 1) * 2, :] = pos_b.astype(jnp.int32).reshape(2, 128)
        carry = carry + jnp.sum(mb, axis=0, keepdims=True)

    loss_ref[0, 0] = jnp.sum(counts * scoresum) * (0.001 * _E / (_T * _T))
    xs_ref[...] = x * score
    offs_ref[...] = offs.astype(jnp.int32)
    cnts_ref[...] = counts.astype(jnp.int32)


_router = pl.pallas_call(
    _router_body,
    out_shape=[
        jax.ShapeDtypeStruct((_T, _D), jnp.float32),
        jax.ShapeDtypeStruct((16, 128), jnp.int32),
        jax.ShapeDtypeStruct((1, _E), jnp.int32),
        jax.ShapeDtypeStruct((1, _E), jnp.int32),
        jax.ShapeDtypeStruct((1, 1), jnp.float32),
    ],
)


_sc_mesh = plsc.VectorSubcoreMesh(core_axis_name="c", subcore_axis_name="s")
_sc_scratch = [
    pltpu.VMEM((_BPW,), jnp.int32),
    pltpu.VMEM((_BPW, _D), jnp.float32),
    pltpu.SemaphoreType.DMA,
]


@functools.partial(
    pl.kernel,
    out_type=jax.ShapeDtypeStruct((_T, _D), jnp.float32),
    mesh=_sc_mesh,
    scratch_types=_sc_scratch,
)
def _sc_scatter_rows(src_hbm, pos_hbm, out_hbm, idx_v, rows_v, sem):
    wid = lax.axis_index("s") * 2 + lax.axis_index("c")
    base = wid * _BPW
    pltpu.sync_copy(pos_hbm.at[pl.ds(base, _BPW)], idx_v)
    pltpu.sync_copy(src_hbm.at[pl.ds(base, _BPW)], rows_v)
    pltpu.async_copy(rows_v, out_hbm.at[idx_v], sem).wait()


@functools.partial(
    pl.kernel,
    out_type=jax.ShapeDtypeStruct((_T, _D), jnp.float32),
    mesh=_sc_mesh,
    scratch_types=_sc_scratch,
)
def _sc_gather_rows(src_hbm, pos_hbm, out_hbm, idx_v, rows_v, sem):
    wid = lax.axis_index("s") * 2 + lax.axis_index("c")
    base = wid * _BPW
    pltpu.sync_copy(pos_hbm.at[pl.ds(base, _BPW)], idx_v)
    pltpu.async_copy(src_hbm.at[idx_v], rows_v, sem).wait()
    pltpu.sync_copy(rows_v, out_hbm.at[pl.ds(base, _BPW)])


def _gmm_body(offs_ref, cnts_ref, x_ref, g_ref, u_ref, d_ref, out_ref):
    e = pl.program_id(0)
    fb = pl.program_id(1)

    @pl.when((e == 0) & (fb == 0))
    def _init():
        out_ref[...] = jnp.zeros_like(out_ref)

    off = offs_ref[e]
    end = off + cnts_ref[e]
    blk0 = (off // _RB) * _RB
    nblk = (end - blk0 + _RB - 1) // _RB
    g = g_ref[0]
    u = u_ref[0]
    d = d_ref[0]

    def body(i, carry):
        s = blk0 + i * _RB
        xb = x_ref[pl.ds(s, _RB), :]
        hg = jnp.dot(xb, g, preferred_element_type=jnp.float32)
        hu = jnp.dot(xb, u, preferred_element_type=jnp.float32)
        h = hg * jax.nn.sigmoid(hg) * hu
        y = jnp.dot(h, d, preferred_element_type=jnp.float32)
        rows = s + lax.broadcasted_iota(jnp.int32, (_RB, 1), 0)
        keep = (rows >= off) & (rows < end)
        out_ref[pl.ds(s, _RB), :] = out_ref[pl.ds(s, _RB), :] + jnp.where(keep, y, 0.0)
        return carry

    lax.fori_loop(0, nblk, body, 0)


_gmm = pl.pallas_call(
    _gmm_body,
    grid_spec=pltpu.PrefetchScalarGridSpec(
        num_scalar_prefetch=2,
        grid=(_E, _NFB),
        in_specs=[
            pl.BlockSpec((_T, _D), lambda e, fb, *_: (0, 0)),
            pl.BlockSpec((1, _D, _FB), lambda e, fb, *_: (e, 0, fb)),
            pl.BlockSpec((1, _D, _FB), lambda e, fb, *_: (e, 0, fb)),
            pl.BlockSpec((1, _FB, _D), lambda e, fb, *_: (e, fb, 0)),
        ],
        out_specs=pl.BlockSpec((_T, _D), lambda e, fb, *_: (0, 0)),
    ),
    out_shape=jax.ShapeDtypeStruct((_T, _D), jnp.float32),
)


_SB = 256


def _shared_body(x_ref, g_ref, u_ref, d_ref, yr_ref, o_ref):
    xb = x_ref[...]
    hg = jnp.dot(xb, g_ref[...], preferred_element_type=jnp.float32)
    hu = jnp.dot(xb, u_ref[...], preferred_element_type=jnp.float32)
    h = hg * jax.nn.sigmoid(hg) * hu
    o_ref[...] = jnp.dot(h, d_ref[...], preferred_element_type=jnp.float32) + yr_ref[...]


_shared = pl.pallas_call(
    _shared_body,
    grid=(_T // _SB,),
    in_specs=[
        pl.BlockSpec((_SB, _D), lambda i: (i, 0)),
        pl.BlockSpec((_D, _F), lambda i: (0, 0)),
        pl.BlockSpec((_D, _F), lambda i: (0, 0)),
        pl.BlockSpec((_F, _D), lambda i: (0, 0)),
        pl.BlockSpec((_SB, _D), lambda i: (i, 0)),
    ],
    out_specs=pl.BlockSpec((_SB, _D), lambda i: (i, 0)),
    out_shape=jax.ShapeDtypeStruct((_T, _D), jnp.float32),
)


def kernel(hidden_states, router_w, gate_w, up_w, down_w,
           shared_gate_w, shared_up_w, shared_down_w):
    xs, pos2d, offs, cnts, loss = _router(hidden_states, router_w)
    pos = pos2d.reshape(_T)
    x_sorted = _sc_scatter_rows(xs, pos)
    y_sorted = _gmm(offs.reshape(_E), cnts.reshape(_E), x_sorted,
                    gate_w, up_w, down_w)
    y_routed = _sc_gather_rows(y_sorted, pos)
    out = _shared(hidden_states, shared_gate_w, shared_up_w, shared_down_w,
                  y_routed)
    return out, loss.reshape(())


# speedup vs baseline: 4.7822x; 4.7822x over previous
import functools

import jax
import jax.numpy as jnp
from jax import lax
from jax.experimental import pallas as pl
from jax.experimental.pallas import tpu as pltpu
from jax.experimental.pallas import tpu_sc as plsc

_T, _D, _F, _E = 2048, 1024, 2048, 64
_RB = 32
_FB = 512
_NFB = _F // _FB
_TB = 256
_NW = 32
_BPW = _T // _NW


def _router_body(x_ref, rw_ref, xs_ref, pos_ref, offs_ref, cnts_ref, loss_ref):
    x = x_ref[...]
    logits = jnp.dot(x, rw_ref[...], preferred_element_type=jnp.float32)
    m = jnp.max(logits, axis=1, keepdims=True)
    eidx = lax.broadcasted_iota(jnp.int32, (_T, _E), 1)
    idx = jnp.min(jnp.where(logits == m, eidx, _E), axis=1, keepdims=True)
    onehot = (eidx == idx).astype(jnp.float32)
    score = jax.nn.sigmoid(m)

    counts = jnp.sum(onehot, axis=0, keepdims=True)
    scoresum = jnp.sum(onehot * score, axis=0, keepdims=True)
    ri = lax.broadcasted_iota(jnp.int32, (_E, _E), 0)
    ci = lax.broadcasted_iota(jnp.int32, (_E, _E), 1)
    offs = jnp.dot(counts, (ri < ci).astype(jnp.float32),
                   preferred_element_type=jnp.float32)

    rt = lax.broadcasted_iota(jnp.int32, (_TB, _TB), 0)
    ct = lax.broadcasted_iota(jnp.int32, (_TB, _TB), 1)
    lstrict = (ct < rt).astype(jnp.float32)
    carry = jnp.zeros((1, _E), jnp.float32)
    for b in range(_T // _TB):
        mb = lax.slice(onehot, (b * _TB, 0), ((b + 1) * _TB, _E))
        rank_b = jnp.dot(lstrict, mb, preferred_element_type=jnp.float32) + carry
        pos_b = jnp.sum(mb * (rank_b + offs), axis=1)
        pos_ref[b * 2:(b + 1) * 2, :] = pos_b.astype(jnp.int32).reshape(2, 128)
        carry = carry + jnp.sum(mb, axis=0, keepdims=True)

    loss_ref[...] = jnp.sum(counts * scoresum).reshape(1, 1) * (0.001 * _E / (_T * _T))
    xs_ref[...] = x * score
    offs_ref[...] = offs.astype(jnp.int32)
    cnts_ref[...] = counts.astype(jnp.int32)


_router = pl.pallas_call(
    _router_body,
    out_shape=[
        jax.ShapeDtypeStruct((_T, _D), jnp.float32),
        jax.ShapeDtypeStruct((16, 128), jnp.int32),
        jax.ShapeDtypeStruct((1, _E), jnp.int32),
        jax.ShapeDtypeStruct((1, _E), jnp.int32),
        jax.ShapeDtypeStruct((1, 1), jnp.float32),
    ],
)


@functools.cache
def _sc_permute_kernels():
    mesh = plsc.VectorSubcoreMesh(core_axis_name="c", subcore_axis_name="s")
    scratch = [
        pltpu.VMEM((_BPW,), jnp.int32),
        pltpu.VMEM((_BPW, _D), jnp.float32),
        pltpu.SemaphoreType.DMA,
    ]

    @functools.partial(
        pl.kernel,
        out_type=jax.ShapeDtypeStruct((_T, _D), jnp.float32),
        mesh=mesh,
        scratch_types=scratch,
    )
    def sc_scatter_rows(src_hbm, pos_hbm, out_hbm, idx_v, rows_v, sem):
        wid = lax.axis_index("s") * 2 + lax.axis_index("c")
        base = wid * _BPW
        pltpu.sync_copy(pos_hbm.at[pl.ds(base, _BPW)], idx_v)
        pltpu.sync_copy(src_hbm.at[pl.ds(base, _BPW)], rows_v)
        pltpu.async_copy(rows_v, out_hbm.at[idx_v], sem).wait()

    @functools.partial(
        pl.kernel,
        out_type=jax.ShapeDtypeStruct((_T, _D), jnp.float32),
        mesh=mesh,
        scratch_types=scratch,
    )
    def sc_gather_rows(src_hbm, pos_hbm, out_hbm, idx_v, rows_v, sem):
        wid = lax.axis_index("s") * 2 + lax.axis_index("c")
        base = wid * _BPW
        pltpu.sync_copy(pos_hbm.at[pl.ds(base, _BPW)], idx_v)
        pltpu.async_copy(src_hbm.at[idx_v], rows_v, sem).wait()
        pltpu.sync_copy(rows_v, out_hbm.at[pl.ds(base, _BPW)])

    return sc_scatter_rows, sc_gather_rows


def _gmm_body(offs_ref, cnts_ref, x_ref, g_ref, u_ref, d_ref, out_ref):
    e = pl.program_id(0)
    fb = pl.program_id(1)

    @pl.when((e == 0) & (fb == 0))
    def _init():
        out_ref[...] = jnp.zeros_like(out_ref)

    off = offs_ref[e]
    end = off + cnts_ref[e]
    blk0 = (off // _RB) * _RB
    nblk = (end - blk0 + _RB - 1) // _RB
    g = g_ref[0]
    u = u_ref[0]
    d = d_ref[0]

    def body(i, carry):
        s = blk0 + i * _RB
        xb = x_ref[pl.ds(s, _RB), :]
        hg = jnp.dot(xb, g, preferred_element_type=jnp.float32)
        hu = jnp.dot(xb, u, preferred_element_type=jnp.float32)
        h = hg * jax.nn.sigmoid(hg) * hu
        y = jnp.dot(h, d, preferred_element_type=jnp.float32)
        rows = s + lax.broadcasted_iota(jnp.int32, (_RB, 1), 0)
        keep = (rows >= off) & (rows < end)
        out_ref[pl.ds(s, _RB), :] = out_ref[pl.ds(s, _RB), :] + jnp.where(keep, y, 0.0)
        return carry

    lax.fori_loop(0, nblk, body, 0)


_gmm = pl.pallas_call(
    _gmm_body,
    grid_spec=pltpu.PrefetchScalarGridSpec(
        num_scalar_prefetch=2,
        grid=(_E, _NFB),
        in_specs=[
            pl.BlockSpec((_T, _D), lambda e, fb, *_: (0, 0)),
            pl.BlockSpec((1, _D, _FB), lambda e, fb, *_: (e, 0, fb)),
            pl.BlockSpec((1, _D, _FB), lambda e, fb, *_: (e, 0, fb)),
            pl.BlockSpec((1, _FB, _D), lambda e, fb, *_: (e, fb, 0)),
        ],
        out_specs=pl.BlockSpec((_T, _D), lambda e, fb, *_: (0, 0)),
    ),
    out_shape=jax.ShapeDtypeStruct((_T, _D), jnp.float32),
)


_SB = 256


def _shared_body(x_ref, g_ref, u_ref, d_ref, yr_ref, o_ref):
    xb = x_ref[...]
    hg = jnp.dot(xb, g_ref[...], preferred_element_type=jnp.float32)
    hu = jnp.dot(xb, u_ref[...], preferred_element_type=jnp.float32)
    h = hg * jax.nn.sigmoid(hg) * hu
    o_ref[...] = jnp.dot(h, d_ref[...], preferred_element_type=jnp.float32) + yr_ref[...]


_shared = pl.pallas_call(
    _shared_body,
    grid=(_T // _SB,),
    in_specs=[
        pl.BlockSpec((_SB, _D), lambda i: (i, 0)),
        pl.BlockSpec((_D, _F), lambda i: (0, 0)),
        pl.BlockSpec((_D, _F), lambda i: (0, 0)),
        pl.BlockSpec((_F, _D), lambda i: (0, 0)),
        pl.BlockSpec((_SB, _D), lambda i: (i, 0)),
    ],
    out_specs=pl.BlockSpec((_SB, _D), lambda i: (i, 0)),
    out_shape=jax.ShapeDtypeStruct((_T, _D), jnp.float32),
)


def kernel(hidden_states, router_w, gate_w, up_w, down_w,
           shared_gate_w, shared_up_w, shared_down_w):
    sc_scatter_rows, sc_gather_rows = _sc_permute_kernels()
    xs, pos2d, offs, cnts, loss = _router(hidden_states, router_w)
    pos = pos2d.reshape(_T)
    x_sorted = sc_scatter_rows(xs, pos)
    y_sorted = _gmm(offs.reshape(_E), cnts.reshape(_E), x_sorted,
                    gate_w, up_w, down_w)
    y_routed = sc_gather_rows(y_sorted, pos)
    out = _shared(hidden_states, shared_gate_w, shared_up_w, shared_down_w,
                  y_routed)
    return out, loss.reshape(())


# speedup vs baseline: 5.4712x; 1.1441x over previous
import functools

import jax
import jax.numpy as jnp
from jax import lax
from jax.experimental import pallas as pl
from jax.experimental.pallas import tpu as pltpu
from jax.experimental.pallas import tpu_sc as plsc

_T, _D, _F, _E = 2048, 1024, 2048, 64
_RB = 32
_FB = 1024
_NFB = _F // _FB
_TB = 256
_NW = 32
_BPW = _T // _NW


def _router_body(x_ref, rw_ref, xs_ref, pos_ref, offs_ref, cnts_ref, loss_ref):
    x = x_ref[...]
    logits = jnp.dot(x, rw_ref[...], preferred_element_type=jnp.float32)
    m = jnp.max(logits, axis=1, keepdims=True)
    eidx = lax.broadcasted_iota(jnp.int32, (_T, _E), 1)
    idx = jnp.min(jnp.where(logits == m, eidx, _E), axis=1, keepdims=True)
    onehot = (eidx == idx).astype(jnp.float32)
    score = jax.nn.sigmoid(m)

    counts = jnp.sum(onehot, axis=0, keepdims=True)
    scoresum = jnp.sum(onehot * score, axis=0, keepdims=True)
    ri = lax.broadcasted_iota(jnp.int32, (_E, _E), 0)
    ci = lax.broadcasted_iota(jnp.int32, (_E, _E), 1)
    offs = jnp.dot(counts, (ri < ci).astype(jnp.float32),
                   preferred_element_type=jnp.float32)

    rt = lax.broadcasted_iota(jnp.int32, (_TB, _TB), 0)
    ct = lax.broadcasted_iota(jnp.int32, (_TB, _TB), 1)
    lstrict = (ct < rt).astype(jnp.float32)
    carry = jnp.zeros((1, _E), jnp.float32)
    for b in range(_T // _TB):
        mb = lax.slice(onehot, (b * _TB, 0), ((b + 1) * _TB, _E))
        rank_b = jnp.dot(lstrict, mb, preferred_element_type=jnp.float32) + carry
        pos_b = jnp.sum(mb * (rank_b + offs), axis=1)
        pos_ref[b * 2:(b + 1) * 2, :] = pos_b.astype(jnp.int32).reshape(2, 128)
        carry = carry + jnp.sum(mb, axis=0, keepdims=True)

    loss_ref[...] = jnp.sum(counts * scoresum).reshape(1, 1) * (0.001 * _E / (_T * _T))
    xs_ref[...] = x * score
    offs_ref[...] = offs.astype(jnp.int32)
    cnts_ref[...] = counts.astype(jnp.int32)


_router = pl.pallas_call(
    _router_body,
    out_shape=[
        jax.ShapeDtypeStruct((_T, _D), jnp.float32),
        jax.ShapeDtypeStruct((16, 128), jnp.int32),
        jax.ShapeDtypeStruct((1, _E), jnp.int32),
        jax.ShapeDtypeStruct((1, _E), jnp.int32),
        jax.ShapeDtypeStruct((1, 1), jnp.float32),
    ],
)


@functools.cache
def _sc_permute_kernels():
    mesh = plsc.VectorSubcoreMesh(core_axis_name="c", subcore_axis_name="s")
    scratch = [
        pltpu.VMEM((_BPW,), jnp.int32),
        pltpu.VMEM((_BPW, _D), jnp.float32),
        pltpu.SemaphoreType.DMA,
    ]

    @functools.partial(
        pl.kernel,
        out_type=jax.ShapeDtypeStruct((_T, _D), jnp.float32),
        mesh=mesh,
        scratch_types=scratch,
    )
    def sc_scatter_rows(src_hbm, pos_hbm, out_hbm, idx_v, rows_v, sem):
        wid = lax.axis_index("s") * 2 + lax.axis_index("c")
        base = wid * _BPW
        pltpu.sync_copy(pos_hbm.at[pl.ds(base, _BPW)], idx_v)
        pltpu.sync_copy(src_hbm.at[pl.ds(base, _BPW)], rows_v)
        pltpu.async_copy(rows_v, out_hbm.at[idx_v], sem).wait()

    @functools.partial(
        pl.kernel,
        out_type=jax.ShapeDtypeStruct((_T, _D), jnp.float32),
        mesh=mesh,
        scratch_types=scratch,
    )
    def sc_gather_rows(src_hbm, pos_hbm, out_hbm, idx_v, rows_v, sem):
        wid = lax.axis_index("s") * 2 + lax.axis_index("c")
        base = wid * _BPW
        pltpu.sync_copy(pos_hbm.at[pl.ds(base, _BPW)], idx_v)
        pltpu.async_copy(src_hbm.at[idx_v], rows_v, sem).wait()
        pltpu.sync_copy(rows_v, out_hbm.at[pl.ds(base, _BPW)])

    return sc_scatter_rows, sc_gather_rows


def _gmm_body(offs_ref, cnts_ref, x_ref, g_ref, u_ref, d_ref, out_ref):
    e = pl.program_id(0)
    fb = pl.program_id(1)

    @pl.when((e == 0) & (fb == 0))
    def _init():
        out_ref[...] = jnp.zeros_like(out_ref)

    off = offs_ref[e]
    end = off + cnts_ref[e]
    blk0 = (off // _RB) * _RB
    nblk = (end - blk0 + _RB - 1) // _RB
    g = g_ref[0]
    u = u_ref[0]
    d = d_ref[0]

    g = g.astype(jnp.bfloat16)
    u = u.astype(jnp.bfloat16)
    d = d.astype(jnp.bfloat16)

    def body(i, carry):
        s = blk0 + i * _RB
        xb = x_ref[pl.ds(s, _RB), :].astype(jnp.bfloat16)
        hg = jnp.dot(xb, g, preferred_element_type=jnp.float32)
        hu = jnp.dot(xb, u, preferred_element_type=jnp.float32)
        h = (hg * jax.nn.sigmoid(hg) * hu).astype(jnp.bfloat16)
        y = jnp.dot(h, d, preferred_element_type=jnp.float32)
        rows = s + lax.broadcasted_iota(jnp.int32, (_RB, 1), 0)
        keep = (rows >= off) & (rows < end)
        out_ref[pl.ds(s, _RB), :] = out_ref[pl.ds(s, _RB), :] + jnp.where(keep, y, 0.0)
        return carry

    lax.fori_loop(0, nblk, body, 0)


_gmm = pl.pallas_call(
    _gmm_body,
    grid_spec=pltpu.PrefetchScalarGridSpec(
        num_scalar_prefetch=2,
        grid=(_E, _NFB),
        in_specs=[
            pl.BlockSpec((_T, _D), lambda e, fb, *_: (0, 0)),
            pl.BlockSpec((1, _D, _FB), lambda e, fb, *_: (e, 0, fb)),
            pl.BlockSpec((1, _D, _FB), lambda e, fb, *_: (e, 0, fb)),
            pl.BlockSpec((1, _FB, _D), lambda e, fb, *_: (e, fb, 0)),
        ],
        out_specs=pl.BlockSpec((_T, _D), lambda e, fb, *_: (0, 0)),
    ),
    out_shape=jax.ShapeDtypeStruct((_T, _D), jnp.float32),
)


_SB = 256


def _shared_body(x_ref, g_ref, u_ref, d_ref, yr_ref, o_ref):
    xb = x_ref[...].astype(jnp.bfloat16)
    g = g_ref[...].astype(jnp.bfloat16)
    u = u_ref[...].astype(jnp.bfloat16)
    hg = jnp.dot(xb, g, preferred_element_type=jnp.float32)
    hu = jnp.dot(xb, u, preferred_element_type=jnp.float32)
    h = (hg * jax.nn.sigmoid(hg) * hu).astype(jnp.bfloat16)
    d = d_ref[...].astype(jnp.bfloat16)
    o_ref[...] = jnp.dot(h, d, preferred_element_type=jnp.float32) + yr_ref[...]


_shared = pl.pallas_call(
    _shared_body,
    grid=(_T // _SB,),
    in_specs=[
        pl.BlockSpec((_SB, _D), lambda i: (i, 0)),
        pl.BlockSpec((_D, _F), lambda i: (0, 0)),
        pl.BlockSpec((_D, _F), lambda i: (0, 0)),
        pl.BlockSpec((_F, _D), lambda i: (0, 0)),
        pl.BlockSpec((_SB, _D), lambda i: (i, 0)),
    ],
    out_specs=pl.BlockSpec((_SB, _D), lambda i: (i, 0)),
    out_shape=jax.ShapeDtypeStruct((_T, _D), jnp.float32),
)


def kernel(hidden_states, router_w, gate_w, up_w, down_w,
           shared_gate_w, shared_up_w, shared_down_w):
    sc_scatter_rows, sc_gather_rows = _sc_permute_kernels()
    xs, pos2d, offs, cnts, loss = _router(hidden_states, router_w)
    pos = pos2d.reshape(_T)
    x_sorted = sc_scatter_rows(xs, pos)
    y_sorted = _gmm(offs.reshape(_E), cnts.reshape(_E), x_sorted,
                    gate_w, up_w, down_w)
    y_routed = sc_gather_rows(y_sorted, pos)
    out = _shared(hidden_states, shared_gate_w, shared_up_w, shared_down_w,
                  y_routed)
    return out, loss.reshape(())
